# Initial kernel scaffold; baseline (speedup 1.0000x reference)
import functools

import jax
import jax.numpy as jnp
from jax import lax
from jax.experimental import pallas as pl
from jax.experimental.pallas import tpu as pltpu
from jax.experimental.pallas import tpu_sc as plsc

_D = 128
_WIN = 100.0
_WAVELENGTH = 0.001
_NC, _NS = 2, 16
_NW = _NC * _NS
_LANES = 16
_CHUNK = 256


def _compute_idx(mass_v, idx_v, n):
    def body(i, carry):
        m = mass_v[pl.ds(i * _LANES, _LANES)]
        m = jnp.minimum(jnp.maximum(m, -_WIN), _WIN)
        m = (m + 1.0) + _WIN
        idx_v[pl.ds(i * _LANES, _LANES)] = (m / _WAVELENGTH).astype(jnp.int32)
        return carry

    lax.fori_loop(0, n // _LANES, body, 0)


def _sc_gather(mass_flat, pe):
    B = mass_flat.shape[0]
    bpw = B // _NW
    nchunk = bpw // _CHUNK
    mesh = plsc.VectorSubcoreMesh(core_axis_name="c", subcore_axis_name="s")

    @functools.partial(
        pl.kernel,
        out_type=jax.ShapeDtypeStruct((B, _D), jnp.float32),
        mesh=mesh,
        scratch_types=[
            pltpu.VMEM((_CHUNK,), jnp.float32),
            pltpu.VMEM((_CHUNK,), jnp.int32),
            pltpu.VMEM((_CHUNK, _D), jnp.float32),
            pltpu.SemaphoreType.DMA,
        ],
    )
    def k(mass_hbm, pe_hbm, out_hbm, mass_v, idx_v, rows_v, sem):
        wid = lax.axis_index("s") * _NC + lax.axis_index("c")
        base = wid * bpw

        def chunk_body(c, carry):
            off = base + c * _CHUNK
            pltpu.sync_copy(mass_hbm.at[pl.ds(off, _CHUNK)], mass_v)
            _compute_idx(mass_v, idx_v, _CHUNK)
            pltpu.async_copy(pe_hbm.at[idx_v], rows_v, sem).wait()
            pltpu.sync_copy(rows_v, out_hbm.at[pl.ds(off, _CHUNK)])
            return carry

        lax.fori_loop(0, nchunk, chunk_body, 0)

    return k(mass_flat, pe)


def kernel(mass, pe):
    b, s, one = mass.shape
    out = _sc_gather(mass.reshape(b * s * one), pe)
    return out.reshape(b, s, one, _D)


# speedup vs baseline: 2.6875x; 2.6875x over previous
import functools

import jax
import jax.numpy as jnp
from jax import lax
from jax.experimental import pallas as pl
from jax.experimental.pallas import tpu as pltpu
from jax.experimental.pallas import tpu_sc as plsc

_D = 128
_WIN = 100.0
_WAVELENGTH = 0.001
_NC, _NS = 2, 16
_NW = _NC * _NS
_LANES = 16
_CHUNK = 256


def _compute_idx(mass_v, idx_v, n):
    def body(i, carry):
        m = mass_v[pl.ds(i * _LANES, _LANES)]
        m = jnp.minimum(jnp.maximum(m, -_WIN), _WIN)
        m = (m + 1.0) + _WIN
        idx_v[pl.ds(i * _LANES, _LANES)] = (m / _WAVELENGTH).astype(jnp.int32)
        return carry

    lax.fori_loop(0, n // _LANES, body, 0)


def _sc_gather(mass_flat, pe):
    B = mass_flat.shape[0]
    bpw = B // _NW
    nchunk = bpw // _CHUNK
    mesh = plsc.VectorSubcoreMesh(
        core_axis_name="c", subcore_axis_name="s",
        num_cores=_NC, num_subcores=_NS,
    )

    @functools.partial(
        pl.kernel,
        out_type=jax.ShapeDtypeStruct((B, _D), jnp.float32),
        mesh=mesh,
        scratch_types=[
            pltpu.VMEM((_CHUNK,), jnp.float32),
            pltpu.VMEM((_CHUNK,), jnp.int32),
            pltpu.VMEM((_CHUNK, _D), jnp.float32),
            pltpu.SemaphoreType.DMA,
        ],
    )
    def k(mass_hbm, pe_hbm, out_hbm, mass_v, idx_v, rows_v, sem):
        wid = lax.axis_index("s") * _NC + lax.axis_index("c")
        base = wid * bpw

        def chunk_body(c, carry):
            off = base + c * _CHUNK
            pltpu.sync_copy(mass_hbm.at[pl.ds(off, _CHUNK)], mass_v)
            _compute_idx(mass_v, idx_v, _CHUNK)
            pltpu.async_copy(pe_hbm.at[idx_v], rows_v, sem).wait()
            pltpu.sync_copy(rows_v, out_hbm.at[pl.ds(off, _CHUNK)])
            return carry

        lax.fori_loop(0, nchunk, chunk_body, 0)

    return k(mass_flat, pe)


def kernel(mass, pe):
    b, s, one = mass.shape
    out = _sc_gather(mass.reshape(b * s * one), pe)
    return out.reshape(b, s, one, _D)
